# baseline (speedup 1.0000x reference)
import functools

import jax
import jax.numpy as jnp
from jax import lax
from jax.experimental import pallas as pl
from jax.experimental.pallas import tpu as pltpu
from jax.experimental.pallas import tpu_sc as plsc

B, L, D = 128, 512, 768
N = B * L
NC, NS, LANES = 2, 16, 16
NW = NC * NS
PER_W = N // NW
C = 32
NCHUNK = PER_W // C
DV = D // LANES


def _body(tok_hbm, pos_hbm, seg_hbm, ttab, ptab, stab, out_hbm,
          tok_idx, pos_idx, seg_idx, trows, prows, srows, sem):
    wid = lax.axis_index("s") * NC + lax.axis_index("c")
    base = wid * PER_W

    def chunk(g, carry):
        off = base + g * C
        pltpu.sync_copy(tok_hbm.at[pl.ds(off, C)], tok_idx)
        pltpu.sync_copy(pos_hbm.at[pl.ds(off, C)], pos_idx)
        pltpu.sync_copy(seg_hbm.at[pl.ds(off, C)], seg_idx)
        pltpu.async_copy(ttab.at[tok_idx], trows, sem).wait()
        pltpu.async_copy(ptab.at[pos_idx], prows, sem).wait()
        pltpu.async_copy(stab.at[seg_idx], srows, sem).wait()

        def addrow(t, carry2):
            for k in range(DV):
                sl = pl.ds(k * LANES, LANES)
                trows[t, sl] = trows[t, sl] + prows[t, sl] + srows[t, sl]
            return carry2

        lax.fori_loop(0, C, addrow, 0)
        pltpu.sync_copy(trows, out_hbm.at[pl.ds(off, C)])
        return carry

    lax.fori_loop(0, NCHUNK, chunk, 0)


@jax.jit
def _gather_sum(tok, pos, seg, ttab, ptab, stab):
    mesh = plsc.VectorSubcoreMesh(core_axis_name="c", subcore_axis_name="s")
    f = functools.partial(
        pl.kernel,
        mesh=mesh,
        out_type=jax.ShapeDtypeStruct((N, D), jnp.float32),
        scratch_types=[
            pltpu.VMEM((C,), jnp.int32),
            pltpu.VMEM((C,), jnp.int32),
            pltpu.VMEM((C,), jnp.int32),
            pltpu.VMEM((C, D), jnp.float32),
            pltpu.VMEM((C, D), jnp.float32),
            pltpu.VMEM((C, D), jnp.float32),
            pltpu.SemaphoreType.DMA,
        ],
    )(_body)
    return f(tok, pos, seg, ttab, ptab, stab)


def kernel(token_ids, position_ids, segment_ids, token_table, position_table, segment_table):
    tok = token_ids.reshape(N).astype(jnp.int32)
    pos = position_ids.reshape(N).astype(jnp.int32)
    seg = segment_ids.reshape(N).astype(jnp.int32)
    out = _gather_sum(tok, pos, seg, token_table, position_table, segment_table)
    return out.reshape(B, L, D)


# speedup vs baseline: 7.4883x; 7.4883x over previous
import functools

import jax
import jax.numpy as jnp
from jax import lax
from jax.experimental import pallas as pl
from jax.experimental.pallas import tpu as pltpu
from jax.experimental.pallas import tpu_sc as plsc

B, L, D = 128, 512, 768
N = B * L
NC, NS, LANES = 2, 16, 16
NW = NC * NS
PER_W = N // NW
C = LANES
NCHUNK = PER_W // C
NBUF = 4
DV = D // LANES


def _prep_body(ptab, stab, pos, seg, comb, cid):
    p = ptab[...]
    comb[pl.ds(0, 512), :] = p + stab[0:1, :]
    comb[pl.ds(512, 512), :] = p + stab[1:2, :]
    cid[...] = seg[...] * 512 + pos[...]


@jax.jit
def _prep(ptab, stab, pos, seg):
    return pl.pallas_call(
        _prep_body,
        out_shape=(
            jax.ShapeDtypeStruct((2 * 512, D), jnp.float32),
            jax.ShapeDtypeStruct((B, L), jnp.int32),
        ),
    )(ptab, stab, pos, seg)


def _sc_body(tok_hbm, cid_hbm, ttab, ctab, out_hbm, *scratch):
    tok_idx, cid_idx = scratch[0], scratch[1]
    bufT = scratch[2:2 + NBUF]
    bufC = scratch[2 + NBUF:2 + 2 * NBUF]
    sem_in = scratch[2 + 2 * NBUF:2 + 3 * NBUF]
    sem_out = scratch[2 + 3 * NBUF:2 + 4 * NBUF]

    wid = lax.axis_index("s") * NC + lax.axis_index("c")
    base = wid * PER_W

    pltpu.sync_copy(tok_hbm.at[pl.ds(base, PER_W)], tok_idx)
    pltpu.sync_copy(cid_hbm.at[pl.ds(base, PER_W)], cid_idx)

    def fire_in(cg, b):
        tvec = tok_idx[pl.ds(cg * C, C)]
        cvec = cid_idx[pl.ds(cg * C, C)]
        pltpu.async_copy(ttab.at[tvec], bufT[b], sem_in[b])
        pltpu.async_copy(ctab.at[cvec], bufC[b], sem_in[b])

    def drain_in(b):
        pltpu.make_async_copy(ttab.at[pl.ds(0, C)], bufT[b], sem_in[b]).wait()
        pltpu.make_async_copy(ctab.at[pl.ds(0, C)], bufC[b], sem_in[b]).wait()

    def fire_out(cg, b):
        pltpu.async_copy(bufT[b], out_hbm.at[pl.ds(base + cg * C, C)], sem_out[b])

    def drain_out(b):
        pltpu.make_async_copy(
            bufT[b], out_hbm.at[pl.ds(0, C)], sem_out[b]).wait()

    fire_in(0, 0)
    fire_in(1, 1)

    def step(q, carry):
        for b in range(NBUF):
            cg = q * NBUF + b
            drain_in(b)

            def add_row(t, carry2):
                for k in range(DV):
                    sl = pl.ds(k * LANES, LANES)
                    bufT[b][t, sl] = bufT[b][t, sl] + bufC[b][t, sl]
                return carry2

            lax.fori_loop(0, C, add_row, 0)
            fire_out(cg, b)

            b2 = (b + 2) % NBUF

            @pl.when(cg >= 2)
            def _():
                drain_out(b2)

            @pl.when(cg + 2 < NCHUNK)
            def _():
                fire_in(cg + 2, b2)
        return carry

    lax.fori_loop(0, NCHUNK // NBUF, step, 0)

    drain_out((NCHUNK - 2) % NBUF)
    drain_out((NCHUNK - 1) % NBUF)


@jax.jit
def _embed_sum(tok, cid, ttab, ctab):
    mesh = plsc.VectorSubcoreMesh(core_axis_name="c", subcore_axis_name="s")
    scratch = [
        pltpu.VMEM((PER_W,), jnp.int32),
        pltpu.VMEM((PER_W,), jnp.int32),
    ]
    scratch += [pltpu.VMEM((C, D), jnp.float32) for _ in range(2 * NBUF)]
    scratch += [pltpu.SemaphoreType.DMA for _ in range(2 * NBUF)]
    f = functools.partial(
        pl.kernel,
        mesh=mesh,
        out_type=jax.ShapeDtypeStruct((N, D), jnp.float32),
        scratch_types=scratch,
    )(_sc_body)
    return f(tok, cid, ttab, ctab)


def kernel(token_ids, position_ids, segment_ids, token_table, position_table, segment_table):
    comb, cid = _prep(position_table, segment_table,
                      position_ids.astype(jnp.int32), segment_ids.astype(jnp.int32))
    tok = token_ids.reshape(N).astype(jnp.int32)
    out = _embed_sum(tok, cid.reshape(N), token_table, comb)
    return out.reshape(B, L, D)
